# baseline (speedup 1.0000x reference)
import functools
import jax
import jax.numpy as jnp
from jax.experimental import pallas as pl

N = 50000
E = 800000
IN_DIM = 10
ED = 9
H = 64
T = 5
OUT = 3

BE = 8000
BN = 1000


def _moments_body(ea_ref, out_ref):
    i = pl.program_id(0)
    blk = ea_ref[...]
    s1 = jnp.sum(blk, axis=0, keepdims=True)
    s2 = jnp.sum(blk * blk, axis=0, keepdims=True)
    acc = jnp.concatenate([s1, s2], axis=0)

    @pl.when(i == 0)
    def _():
        out_ref[...] = acc

    @pl.when(i > 0)
    def _():
        out_ref[...] += acc


def _moments(edge_attr):
    return pl.pallas_call(
        _moments_body,
        grid=(E // BE,),
        in_specs=[pl.BlockSpec((BE, ED), lambda i: (i, 0))],
        out_specs=pl.BlockSpec((2, ED), lambda i: (0, 0)),
        out_shape=jax.ShapeDtypeStruct((2, ED), jnp.float32),
    )(edge_attr)


def _lift_body(x_ref, w1_ref, b1_ref, w2_ref, b2_ref, out_ref):
    h = jnp.maximum(
        jnp.dot(x_ref[...], w1_ref[...], preferred_element_type=jnp.float32)
        + b1_ref[...], 0.0)
    out_ref[...] = (
        jnp.dot(h, w2_ref[...], preferred_element_type=jnp.float32)
        + b2_ref[...])


def _lift(x, w1, b1, w2, b2):
    return pl.pallas_call(
        _lift_body,
        grid=(N // BN,),
        in_specs=[
            pl.BlockSpec((BN, IN_DIM), lambda i: (i, 0)),
            pl.BlockSpec((IN_DIM, H), lambda i: (0, 0)),
            pl.BlockSpec((1, H), lambda i: (0, 0)),
            pl.BlockSpec((H, H), lambda i: (0, 0)),
            pl.BlockSpec((1, H), lambda i: (0, 0)),
        ],
        out_specs=pl.BlockSpec((BN, H), lambda i: (i, 0)),
        out_shape=jax.ShapeDtypeStruct((N, H), jnp.float32),
    )(x, w1, b1.reshape(1, H), w2, b2.reshape(1, H))


def _edge_mlp_body(ea_ref, w1_ref, b1_ref, w2_ref, b2_ref, w3_ref, b3_ref,
                   out_ref):
    ea = ea_ref[...]
    k1 = jnp.maximum(
        jnp.dot(ea, w1_ref[0], preferred_element_type=jnp.float32)
        + b1_ref[0], 0.0)
    k2 = jnp.maximum(
        jnp.dot(k1, w2_ref[0], preferred_element_type=jnp.float32)
        + b2_ref[0], 0.0)
    out_ref[0] = (
        jnp.dot(k2, w3_ref[0], preferred_element_type=jnp.float32)
        + b3_ref[0])


def _edge_mlp(edge_attr, w1e, b1e, w2, b2, w3, b3):
    return pl.pallas_call(
        _edge_mlp_body,
        grid=(T, E // BE),
        in_specs=[
            pl.BlockSpec((BE, ED), lambda t, i: (i, 0)),
            pl.BlockSpec((1, ED, H), lambda t, i: (t, 0, 0)),
            pl.BlockSpec((1, 1, H), lambda t, i: (t, 0, 0)),
            pl.BlockSpec((1, H, H), lambda t, i: (t, 0, 0)),
            pl.BlockSpec((1, 1, H), lambda t, i: (t, 0, 0)),
            pl.BlockSpec((1, H, H), lambda t, i: (t, 0, 0)),
            pl.BlockSpec((1, 1, H), lambda t, i: (t, 0, 0)),
        ],
        out_specs=pl.BlockSpec((1, BE, H), lambda t, i: (t, i, 0)),
        out_shape=jax.ShapeDtypeStruct((T, E, H), jnp.float32),
    )(edge_attr, w1e, b1e.reshape(T, 1, H), w2, b2.reshape(T, 1, H),
      w3, b3.reshape(T, 1, H))


def _node_body(final, v_ref, v0_ref, agg_ref, cnt_ref, x_ref, sw_ref, sb_ref,
               gw_ref, gb_ref, pw1_ref, pb1_ref, pw2_ref, pb2_ref, out_ref):
    cnt = jnp.clip(cnt_ref[...], 1.0, None)
    gate = jax.nn.sigmoid(x_ref[:, 4:5] * gw_ref[...] + gb_ref[...])
    vn = jnp.maximum(
        jnp.dot(v_ref[...], sw_ref[...], preferred_element_type=jnp.float32)
        + sb_ref[...] + agg_ref[...] / cnt + gate * v0_ref[...], 0.0)
    if final:
        h = jnp.maximum(
            jnp.dot(vn, pw1_ref[...], preferred_element_type=jnp.float32)
            + pb1_ref[...], 0.0)
        out_ref[...] = (jnp.dot(h, pw2_ref[...],
                                preferred_element_type=jnp.float32)
                        + pb2_ref[...])
    else:
        out_ref[...] = vn


def _node_update(final, v, v0, agg, cnt, x, sw, sb, gw, gb, pw1, pb1, pw2, pb2):
    if final:
        out_spec = pl.BlockSpec((BN, OUT), lambda i: (i, 0))
        out_shape = jax.ShapeDtypeStruct((N, OUT), jnp.float32)
    else:
        out_spec = pl.BlockSpec((BN, H), lambda i: (i, 0))
        out_shape = jax.ShapeDtypeStruct((N, H), jnp.float32)
    return pl.pallas_call(
        functools.partial(_node_body, final),
        grid=(N // BN,),
        in_specs=[
            pl.BlockSpec((BN, H), lambda i: (i, 0)),
            pl.BlockSpec((BN, H), lambda i: (i, 0)),
            pl.BlockSpec((BN, H), lambda i: (i, 0)),
            pl.BlockSpec((BN, 1), lambda i: (i, 0)),
            pl.BlockSpec((BN, IN_DIM), lambda i: (i, 0)),
            pl.BlockSpec((H, H), lambda i: (0, 0)),
            pl.BlockSpec((1, H), lambda i: (0, 0)),
            pl.BlockSpec((1, H), lambda i: (0, 0)),
            pl.BlockSpec((1, H), lambda i: (0, 0)),
            pl.BlockSpec((H, H), lambda i: (0, 0)),
            pl.BlockSpec((1, H), lambda i: (0, 0)),
            pl.BlockSpec((H, OUT), lambda i: (0, 0)),
            pl.BlockSpec((1, OUT), lambda i: (0, 0)),
        ],
        out_specs=out_spec,
        out_shape=out_shape,
    )(v, v0, agg, cnt, x, sw, sb.reshape(1, H), gw.reshape(1, H),
      gb.reshape(1, H), pw1, pb1.reshape(1, H), pw2, pb2.reshape(1, OUT))


def kernel(x, edge_index, edge_attr, lift_W1, lift_b1, lift_W2, lift_b2,
           ker_W1, ker_b1, ker_W2, ker_b2, ker_W3, ker_b3,
           self_W, self_b, gate_W, gate_b, proj_W1, proj_b1, proj_W2, proj_b2):
    src = edge_index[0]
    dst = edge_index[1]

    mom = _moments(edge_attr)
    mu = mom[0] / E
    var = jnp.maximum(mom[1] / E - mu * mu, 0.0)
    inv = 1.0 / (jnp.sqrt(var) + 1e-6)
    w1e = ker_W1 * inv[None, :, None]
    b1e = ker_b1 - jnp.einsum('d,tdh->th', mu * inv, ker_W1)

    v0 = _lift(x, lift_W1, lift_b1, lift_W2, lift_b2)
    kmats = _edge_mlp(edge_attr, w1e, b1e, ker_W2, ker_b2, ker_W3, ker_b3)

    ones_e = jnp.ones((E,), jnp.float32)
    cnt = jax.ops.segment_sum(ones_e, dst, num_segments=N)[:, None]

    v = v0
    out = None
    for t in range(T):
        m = kmats[t] * jnp.take(v, src, axis=0)
        agg = jax.ops.segment_sum(m, dst, num_segments=N)
        final = (t == T - 1)
        res = _node_update(final, v, v0, agg, cnt, x,
                           self_W[t], self_b[t], gate_W[t], gate_b[t],
                           proj_W1, proj_b1, proj_W2, proj_b2)
        if final:
            out = res
        else:
            v = res
    return out


# speedup vs baseline: 1.0051x; 1.0051x over previous
import functools
import jax
import jax.numpy as jnp
from jax.experimental import pallas as pl

N = 50000
E = 800000
IN_DIM = 10
ED = 9
H = 64
T = 5
OUT = 3

BE = 8000
BN = 1000


def _moments_body(ea_ref, out_ref):
    i = pl.program_id(0)
    blk = ea_ref[...]
    s1 = jnp.sum(blk, axis=0, keepdims=True)
    s2 = jnp.sum(blk * blk, axis=0, keepdims=True)
    acc = jnp.concatenate([s1, s2], axis=0)

    @pl.when(i == 0)
    def _():
        out_ref[...] = acc

    @pl.when(i > 0)
    def _():
        out_ref[...] += acc


def _moments(edge_attr):
    return pl.pallas_call(
        _moments_body,
        grid=(E // BE,),
        in_specs=[pl.BlockSpec((BE, ED), lambda i: (i, 0))],
        out_specs=pl.BlockSpec((2, ED), lambda i: (0, 0)),
        out_shape=jax.ShapeDtypeStruct((2, ED), jnp.float32),
    )(edge_attr)


def _lift_body(x_ref, w1_ref, b1_ref, w2_ref, b2_ref, out_ref):
    h = jnp.maximum(
        jnp.dot(x_ref[...], w1_ref[...], preferred_element_type=jnp.float32)
        + b1_ref[...], 0.0)
    out_ref[...] = (
        jnp.dot(h, w2_ref[...], preferred_element_type=jnp.float32)
        + b2_ref[...])


def _lift(x, w1, b1, w2, b2):
    return pl.pallas_call(
        _lift_body,
        grid=(N // BN,),
        in_specs=[
            pl.BlockSpec((BN, IN_DIM), lambda i: (i, 0)),
            pl.BlockSpec((IN_DIM, H), lambda i: (0, 0)),
            pl.BlockSpec((1, H), lambda i: (0, 0)),
            pl.BlockSpec((H, H), lambda i: (0, 0)),
            pl.BlockSpec((1, H), lambda i: (0, 0)),
        ],
        out_specs=pl.BlockSpec((BN, H), lambda i: (i, 0)),
        out_shape=jax.ShapeDtypeStruct((N, H), jnp.float32),
    )(x, w1, b1.reshape(1, H), w2, b2.reshape(1, H))


def _edge_mlp_body(ea_ref, w1_ref, b1_ref, w2_ref, b2_ref, w3_ref, b3_ref,
                   out_ref):
    ea = ea_ref[...]
    k1 = jnp.maximum(
        jnp.dot(ea, w1_ref[0], preferred_element_type=jnp.float32)
        + b1_ref[0], 0.0)
    k2 = jnp.maximum(
        jnp.dot(k1, w2_ref[0], preferred_element_type=jnp.float32)
        + b2_ref[0], 0.0)
    out_ref[0] = (
        jnp.dot(k2, w3_ref[0], preferred_element_type=jnp.float32)
        + b3_ref[0])


def _edge_mlp(edge_attr, w1e, b1e, w2, b2, w3, b3):
    return pl.pallas_call(
        _edge_mlp_body,
        grid=(T, E // BE),
        in_specs=[
            pl.BlockSpec((BE, ED), lambda t, i: (i, 0)),
            pl.BlockSpec((1, ED, H), lambda t, i: (t, 0, 0)),
            pl.BlockSpec((1, 1, H), lambda t, i: (t, 0, 0)),
            pl.BlockSpec((1, H, H), lambda t, i: (t, 0, 0)),
            pl.BlockSpec((1, 1, H), lambda t, i: (t, 0, 0)),
            pl.BlockSpec((1, H, H), lambda t, i: (t, 0, 0)),
            pl.BlockSpec((1, 1, H), lambda t, i: (t, 0, 0)),
        ],
        out_specs=pl.BlockSpec((1, BE, H), lambda t, i: (t, i, 0)),
        out_shape=jax.ShapeDtypeStruct((T, E, H), jnp.float32),
    )(edge_attr, w1e, b1e.reshape(T, 1, H), w2, b2.reshape(T, 1, H),
      w3, b3.reshape(T, 1, H))


def _node_body(final, v_ref, v0_ref, agg_ref, cnt_ref, x_ref, sw_ref, sb_ref,
               gw_ref, gb_ref, pw1_ref, pb1_ref, pw2_ref, pb2_ref, out_ref):
    cnt = jnp.clip(cnt_ref[...], 1.0, None)
    gate = jax.nn.sigmoid(x_ref[:, 4:5] * gw_ref[...] + gb_ref[...])
    vn = jnp.maximum(
        jnp.dot(v_ref[...], sw_ref[...], preferred_element_type=jnp.float32)
        + sb_ref[...] + agg_ref[...] / cnt + gate * v0_ref[...], 0.0)
    if final:
        h = jnp.maximum(
            jnp.dot(vn, pw1_ref[...], preferred_element_type=jnp.float32)
            + pb1_ref[...], 0.0)
        out_ref[...] = (jnp.dot(h, pw2_ref[...],
                                preferred_element_type=jnp.float32)
                        + pb2_ref[...])
    else:
        out_ref[...] = vn


def _node_update(final, v, v0, agg, cnt, x, sw, sb, gw, gb, pw1, pb1, pw2, pb2):
    if final:
        out_spec = pl.BlockSpec((BN, OUT), lambda i: (i, 0))
        out_shape = jax.ShapeDtypeStruct((N, OUT), jnp.float32)
    else:
        out_spec = pl.BlockSpec((BN, H), lambda i: (i, 0))
        out_shape = jax.ShapeDtypeStruct((N, H), jnp.float32)
    return pl.pallas_call(
        functools.partial(_node_body, final),
        grid=(N // BN,),
        in_specs=[
            pl.BlockSpec((BN, H), lambda i: (i, 0)),
            pl.BlockSpec((BN, H), lambda i: (i, 0)),
            pl.BlockSpec((BN, H), lambda i: (i, 0)),
            pl.BlockSpec((BN, 1), lambda i: (i, 0)),
            pl.BlockSpec((BN, IN_DIM), lambda i: (i, 0)),
            pl.BlockSpec((H, H), lambda i: (0, 0)),
            pl.BlockSpec((1, H), lambda i: (0, 0)),
            pl.BlockSpec((1, H), lambda i: (0, 0)),
            pl.BlockSpec((1, H), lambda i: (0, 0)),
            pl.BlockSpec((H, H), lambda i: (0, 0)),
            pl.BlockSpec((1, H), lambda i: (0, 0)),
            pl.BlockSpec((H, OUT), lambda i: (0, 0)),
            pl.BlockSpec((1, OUT), lambda i: (0, 0)),
        ],
        out_specs=out_spec,
        out_shape=out_shape,
    )(v, v0, agg, cnt, x, sw, sb.reshape(1, H), gw.reshape(1, H),
      gb.reshape(1, H), pw1, pb1.reshape(1, H), pw2, pb2.reshape(1, OUT))


def kernel(x, edge_index, edge_attr, lift_W1, lift_b1, lift_W2, lift_b2,
           ker_W1, ker_b1, ker_W2, ker_b2, ker_W3, ker_b3,
           self_W, self_b, gate_W, gate_b, proj_W1, proj_b1, proj_W2, proj_b2):
    perm = jnp.argsort(edge_index[1])
    src = edge_index[0][perm]
    dst = edge_index[1][perm]
    edge_attr_s = edge_attr[perm]

    mom = _moments(edge_attr)
    mu = mom[0] / E
    var = jnp.maximum(mom[1] / E - mu * mu, 0.0)
    inv = 1.0 / (jnp.sqrt(var) + 1e-6)
    w1e = ker_W1 * inv[None, :, None]
    b1e = ker_b1 - jnp.einsum('d,tdh->th', mu * inv, ker_W1)

    v0 = _lift(x, lift_W1, lift_b1, lift_W2, lift_b2)
    kmats = _edge_mlp(edge_attr_s, w1e, b1e, ker_W2, ker_b2, ker_W3, ker_b3)

    ones_e = jnp.ones((E,), jnp.float32)
    cnt = jax.ops.segment_sum(ones_e, dst, num_segments=N,
                              indices_are_sorted=True)[:, None]

    v = v0
    out = None
    for t in range(T):
        m = kmats[t] * jnp.take(v, src, axis=0)
        agg = jax.ops.segment_sum(m, dst, num_segments=N,
                                  indices_are_sorted=True)
        final = (t == T - 1)
        res = _node_update(final, v, v0, agg, cnt, x,
                           self_W[t], self_b[t], gate_W[t], gate_b[t],
                           proj_W1, proj_b1, proj_W2, proj_b2)
        if final:
            out = res
        else:
            v = res
    return out
